# baseline (speedup 1.0000x reference)
import functools

import jax
import jax.numpy as jnp
from jax import lax
from jax.experimental import pallas as pl
from jax.experimental.pallas import tpu as pltpu
from jax.experimental.pallas import tpu_sc as plsc

N = 50000
E = 800000
HID = 128
NP = 50176
NTILES = 32
EPT = E // NTILES
TBLK = 7168
TSTEPS = NP // TBLK

_mesh = plsc.VectorSubcoreMesh(core_axis_name="c", subcore_axis_name="s")
_sc_params = pltpu.CompilerParams(use_tc_tiling_on_sc=False)


def _deg_body(ei_ref, zeros_ref, ones_ref, out0_ref, out1_ref,
              deg_sp, dbuf, ones_v):
    c = lax.axis_index("c")
    s = lax.axis_index("s")
    wid = c * 16 + s

    @pl.when(s == 0)
    def _():
        pltpu.sync_copy(zeros_ref, deg_sp)

    pltpu.sync_copy(ones_ref, ones_v)
    pltpu.sync_copy(ei_ref.at[1, pl.ds(wid * EPT, EPT)], dbuf)
    plsc.subcore_barrier()
    pltpu.sync_copy(ones_v, deg_sp.at[dbuf], add=True)
    plsc.subcore_barrier()

    @pl.when((s == 0) & (c == 0))
    def _():
        pltpu.sync_copy(deg_sp, out0_ref)

    @pl.when((s == 0) & (c == 1))
    def _():
        pltpu.sync_copy(deg_sp, out1_ref)


_deg_call = functools.partial(
    pl.kernel,
    out_type=[
        jax.ShapeDtypeStruct((NP,), jnp.float32),
        jax.ShapeDtypeStruct((NP,), jnp.float32),
    ],
    mesh=_mesh,
    compiler_params=_sc_params,
    scratch_types=[
        pltpu.VMEM_SHARED((NP,), jnp.float32),
        pltpu.VMEM((EPT,), jnp.int32),
        pltpu.VMEM((EPT,), jnp.float32),
    ],
)(_deg_body)


def _agg_body(ei_ref, g0_ref, g1_ref, zeros_ref,
              o00_ref, o01_ref, o10_ref, o11_ref,
              acc0_sp, acc1_sp, sbuf, dbuf, vals0, vals1, sem0, sem1):
    c = lax.axis_index("c")
    s = lax.axis_index("s")
    wid = c * 16 + s

    @pl.when(s == 0)
    def _():
        pltpu.sync_copy(zeros_ref, acc0_sp)
        pltpu.sync_copy(zeros_ref, acc1_sp)

    base = wid * EPT
    cps = pltpu.async_copy(ei_ref.at[0, pl.ds(base, EPT)], sbuf, sem0)
    cpd = pltpu.async_copy(ei_ref.at[1, pl.ds(base, EPT)], dbuf, sem1)
    cps.wait()
    cp0 = pltpu.async_copy(g0_ref.at[sbuf], vals0, sem0)
    cp1 = pltpu.async_copy(g1_ref.at[sbuf], vals1, sem1)
    cpd.wait()
    plsc.subcore_barrier()
    cp0.wait()
    pltpu.sync_copy(vals0, acc0_sp.at[dbuf], add=True)
    cp1.wait()
    pltpu.sync_copy(vals1, acc1_sp.at[dbuf], add=True)
    plsc.subcore_barrier()

    @pl.when((s == 0) & (c == 0))
    def _():
        pltpu.sync_copy(acc0_sp, o00_ref)
        pltpu.sync_copy(acc1_sp, o01_ref)

    @pl.when((s == 0) & (c == 1))
    def _():
        pltpu.sync_copy(acc0_sp, o10_ref)
        pltpu.sync_copy(acc1_sp, o11_ref)


_agg_call = functools.partial(
    pl.kernel,
    out_type=[
        jax.ShapeDtypeStruct((NP,), jnp.float32),
        jax.ShapeDtypeStruct((NP,), jnp.float32),
        jax.ShapeDtypeStruct((NP,), jnp.float32),
        jax.ShapeDtypeStruct((NP,), jnp.float32),
    ],
    mesh=_mesh,
    compiler_params=_sc_params,
    scratch_types=[
        pltpu.VMEM_SHARED((NP,), jnp.float32),
        pltpu.VMEM_SHARED((NP,), jnp.float32),
        pltpu.VMEM((EPT,), jnp.int32),
        pltpu.VMEM((EPT,), jnp.int32),
        pltpu.VMEM((EPT,), jnp.float32),
        pltpu.VMEM((EPT,), jnp.float32),
        pltpu.SemaphoreType.DMA,
        pltpu.SemaphoreType.DMA,
    ],
)(_agg_body)


def _prep_body(d0_ref, d1_ref, f0_ref, f1_ref, dinv_ref, g0_ref, g1_ref):
    deg = d0_ref[...] + d1_ref[...] + 1.0
    dv = lax.rsqrt(deg)
    dinv_ref[...] = dv
    g0_ref[...] = f0_ref[...] * dv
    g1_ref[...] = f1_ref[...] * dv


_prep_call = pl.pallas_call(
    _prep_body,
    out_shape=[
        jax.ShapeDtypeStruct((NP,), jnp.float32),
        jax.ShapeDtypeStruct((NP,), jnp.float32),
        jax.ShapeDtypeStruct((NP,), jnp.float32),
    ],
)


def _head_body(o00_ref, o10_ref, o01_ref, o11_ref,
               dinv_ref, g0_ref, g1_ref, wgt_ref, bg_ref,
               lnw_ref, lnb_ref, esn_ref, w1t_ref, b1_ref, w2t_ref, b2_ref,
               out_ref, ssum, ssq):
    i = pl.program_id(0)

    @pl.when(i == 0)
    def _():
        ssum[...] = jnp.zeros((HID, 1), jnp.float32)
        ssq[...] = jnp.zeros((HID, 1), jnp.float32)

    a0 = dinv_ref[...] * (o00_ref[...] + o10_ref[...] + g0_ref[...])
    a1 = dinv_ref[...] * (o01_ref[...] + o11_ref[...] + g1_ref[...])
    amat = jnp.concatenate([a0.reshape(1, TBLK), a1.reshape(1, TBLK)], axis=0)
    xt = jnp.dot(wgt_ref[...], amat, preferred_element_type=jnp.float32)
    xt = jnp.maximum(xt + bg_ref[...], 0.0)
    node = i * TBLK + lax.broadcasted_iota(jnp.int32, (TBLK, 1), 0)
    mcol = (node < N).astype(jnp.float32)
    ssum[...] += jnp.dot(xt, mcol, preferred_element_type=jnp.float32)
    ssq[...] += jnp.dot(xt * xt, mcol, preferred_element_type=jnp.float32)

    @pl.when(i == TSTEPS - 1)
    def _():
        s_ch = ssum[...]
        s1 = jnp.sum(s_ch)
        s2 = jnp.sum(ssq[...])
        cnt = float(N) * float(HID)
        mean = s1 / cnt
        std = jnp.sqrt(s2 / cnt - mean * mean)
        pooled = ((s_ch - float(N) * mean) / (std + 1e-5) * lnw_ref[...]
                  + float(N) * lnb_ref[...])
        zt = jnp.concatenate([pooled, esn_ref[...]], axis=0)
        z1 = jnp.dot(w1t_ref[...], zt, preferred_element_type=jnp.float32)
        z1 = jnp.maximum(z1 + b1_ref[...], 0.0)
        lg = jnp.dot(w2t_ref[...], z1, preferred_element_type=jnp.float32)
        lg = lg + b2_ref[...]
        mx = jnp.max(lg, axis=0, keepdims=True)
        e = jnp.exp(lg - mx)
        out_ref[...] = lg - mx - jnp.log(jnp.sum(e, axis=0, keepdims=True))


_blk1 = pl.BlockSpec((TBLK,), lambda i: (i,))
_whole = lambda shape: pl.BlockSpec(shape, lambda i: tuple(0 for _ in shape))

_head_call = pl.pallas_call(
    _head_body,
    grid=(TSTEPS,),
    in_specs=[
        _blk1, _blk1, _blk1, _blk1, _blk1, _blk1, _blk1,
        _whole((HID, 2)),
        _whole((HID, 1)),
        _whole((HID, 1)),
        _whole((HID, 1)),
        _whole((500, 1)),
        _whole((HID, HID + 500)),
        _whole((HID, 1)),
        _whole((64, HID)),
        _whole((64, 1)),
    ],
    out_specs=pl.BlockSpec((64, 1), lambda i: (0, 0)),
    out_shape=jax.ShapeDtypeStruct((64, 1), jnp.float32),
    scratch_shapes=[
        pltpu.VMEM((HID, 1), jnp.float32),
        pltpu.VMEM((HID, 1), jnp.float32),
    ],
)


def kernel(node_feats, edge_index, esn_state, W_gcn, b_gcn, ln_w, ln_b,
           W1, b1, W2, b2):
    zeros1 = jnp.zeros((NP,), jnp.float32)
    ones_e = jnp.ones((EPT,), jnp.float32)

    deg0, deg1 = _deg_call(edge_index, zeros1, ones_e)

    ftp = jnp.pad(node_feats, ((0, NP - N), (0, 0))).T
    dinv, g0, g1 = _prep_call(deg0, deg1, ftp[0], ftp[1])

    o00, o01, o10, o11 = _agg_call(edge_index, g0, g1, zeros1)

    out_col = _head_call(
        o00, o10, o01, o11, dinv, g0, g1,
        W_gcn.T,
        b_gcn.reshape(HID, 1),
        ln_w.reshape(HID, 1), ln_b.reshape(HID, 1),
        esn_state.reshape(500, 1),
        W1.T,
        b1.reshape(HID, 1),
        W2.T,
        b2.reshape(64, 1))
    return out_col.reshape(1, 64)
